# baseline (speedup 1.0000x reference)
import functools

import jax
import jax.numpy as jnp
import numpy as np
from jax import lax
from jax.experimental import pallas as pl
from jax.experimental.pallas import tpu as pltpu
from jax.experimental.pallas import tpu_sc as plsc

NUM_FEATS = 26
SLOTS = 4
COLS = NUM_FEATS * SLOTS
COLS_PAD = 112
VOCAB = 1000
TAB_ROWS = NUM_FEATS * (VOCAB + 1)
EMB = 64
B = 4096
IN_DIM = NUM_FEATS * EMB
OUT_DIM = 4096
NUM_TOKENS = 8
D_MODEL = 512

NW = 32
NSPLIT = 8
BH = B // NSPLIT
ROWS_PER_W = BH // NW
CB = 8
NCHUNK = ROWS_PER_W // CB
JT = IN_DIM // 128

L = 16


def _sc_gather_body(ints_hbm, table_hbm, out_hbm, ints_v, idx_v, off_v, rows_v,
                    out_v, sems):
    cid = lax.axis_index("c")
    sid = lax.axis_index("s")
    wid = sid * 2 + cid
    lane = lax.iota(jnp.int32, L)
    colsv = jnp.full((L,), COLS, jnp.int32)

    for t in range(CB * COLS // L):
        j = lax.rem(t * L + lane, colsv)
        off_v[pl.ds(t * L, L)] = (j >> 2) * (VOCAB + 1)

    def fire(ch, buf):
        base = wid * ROWS_PER_W + ch * CB
        pltpu.sync_copy(ints_hbm.at[pl.ds(base * COLS, CB * COLS)],
                        ints_v.at[buf])
        for t in range(CB * COLS // L):
            sl = pl.ds(t * L, L)
            idx_v[buf, sl] = jnp.clip(ints_v[buf, sl], 0, VOCAB) + off_v[sl]
        pltpu.async_copy(table_hbm.at[idx_v.at[buf]],
                         rows_v.at[buf], sems.at[buf])

    def drain_reduce(ch, buf):
        pltpu.make_async_copy(table_hbm.at[idx_v.at[buf]],
                              rows_v.at[buf], sems.at[buf]).wait()

        def red_b(b, bcarry):
            def red_fp(fp, fcarry):
                for half in range(2):
                    rbase = b * COLS + 8 * fp + 4 * half
                    for c in range(EMB // L):
                        sl = pl.ds(c * L, L)
                        acc = (rows_v[buf, rbase, sl]
                               + rows_v[buf, rbase + 1, sl]
                               + rows_v[buf, rbase + 2, sl]
                               + rows_v[buf, rbase + 3, sl])
                        out_v[fp, b, pl.ds(64 * half + c * L, L)] = acc * 0.25
                return fcarry
            return lax.fori_loop(0, JT, red_fp, bcarry)

        lax.fori_loop(0, CB, red_b, 0)
        pltpu.sync_copy(out_v, out_hbm.at[wid * NCHUNK + ch])

    fire(0, 0)

    def chunk_body(ch, carry):
        buf = lax.rem(ch, 2)

        @pl.when(ch + 1 < NCHUNK)
        def _():
            fire(ch + 1, 1 - buf)

        drain_reduce(ch, buf)
        return carry

    lax.fori_loop(0, NCHUNK, chunk_body, 0)


@jax.jit
def _sc_gather(ints_flat, table_flat):
    mesh = plsc.VectorSubcoreMesh(core_axis_name="c", subcore_axis_name="s")
    return pl.kernel(
        _sc_gather_body,
        out_type=jax.ShapeDtypeStruct((BH // CB, JT, CB, 128), jnp.float32),
        mesh=mesh,
        scratch_types=[
            pltpu.VMEM((2, CB * COLS), jnp.int32),
            pltpu.VMEM((2, CB * COLS), jnp.int32),
            pltpu.VMEM((CB * COLS,), jnp.int32),
            pltpu.VMEM((2, CB * COLS, EMB), jnp.float32),
            pltpu.VMEM((JT, CB, 128), jnp.float32),
            pltpu.SemaphoreType.DMA((2,)),
        ],
        compiler_params=pltpu.CompilerParams(use_tc_tiling_on_sc=False),
    )(ints_flat, table_flat)


BT = 512
NT = 512
N_STEPS = OUT_DIM // NT


def _dense_body_alias(x_ref, w_ref, b_ref, g_ref, be_ref, h_ref, o_ref, acc_ref):
    _dense_body(x_ref, w_ref, b_ref, g_ref, be_ref, o_ref, acc_ref)


def _dense_body(x_ref, w_ref, b_ref, g_ref, be_ref, o_ref, acc_ref):
    n = pl.program_id(1)
    x = jnp.concatenate(
        [x_ref[:, j].reshape(BT, 128) for j in range(JT)], axis=1)
    h = jnp.dot(x.astype(jnp.bfloat16), w_ref[...],
                preferred_element_type=jnp.float32)
    h = h + b_ref[...]
    h = h * jax.nn.sigmoid(h)
    acc_ref[:, pl.ds(n * NT, NT)] = h

    @pl.when(n == N_STEPS - 1)
    def _():
        a = acc_ref[...]
        mu = jnp.mean(a, axis=1, keepdims=True)
        d = a - mu
        var = jnp.mean(d * d, axis=1, keepdims=True)
        out = d * lax.rsqrt(var + 1e-5) * g_ref[...] + be_ref[...]
        o_ref[...] = out.reshape(BT, NUM_TOKENS, D_MODEL)


def _make_tc_dense(tile_off, aliased):
    in_specs = [
        pl.BlockSpec((BT // CB, JT, CB, 128), lambda b, n: (b, 0, 0, 0)),
        pl.BlockSpec((IN_DIM, NT), lambda b, n: (0, n)),
        pl.BlockSpec((1, NT), lambda b, n: (0, n)),
        pl.BlockSpec((1, OUT_DIM), lambda b, n: (0, 0)),
        pl.BlockSpec((1, OUT_DIM), lambda b, n: (0, 0)),
    ]
    if aliased:
        in_specs.append(pl.BlockSpec(memory_space=pl.ANY))

    @jax.jit
    def call(*args):
        return pl.pallas_call(
            _dense_body_alias if aliased else _dense_body,
            grid=(BH // BT, N_STEPS),
            in_specs=in_specs,
            out_specs=pl.BlockSpec((BT, NUM_TOKENS, D_MODEL),
                                   lambda b, n: (b + tile_off, 0, 0)),
            out_shape=jax.ShapeDtypeStruct((B, NUM_TOKENS, D_MODEL),
                                           jnp.float32),
            scratch_shapes=[pltpu.VMEM((BT, OUT_DIM), jnp.float32)],
            input_output_aliases={5: 0} if aliased else {},
            compiler_params=pltpu.CompilerParams(
                dimension_semantics=("parallel", "arbitrary"),
            ),
        )(*args)

    return call


_TC_CALLS = [_make_tc_dense(i * (BH // BT), aliased=(i > 0))
             for i in range(NSPLIT)]


def kernel(int_feats, emb_tables, W1, b1, ln_gamma, ln_beta):
    ints = int_feats.astype(jnp.int32).reshape(NSPLIT, BH * COLS)
    table = emb_tables.reshape(TAB_ROWS, EMB)
    w_bf = W1.astype(jnp.bfloat16)
    b1r = b1.reshape(1, OUT_DIM)
    g = ln_gamma.reshape(1, OUT_DIM)
    be = ln_beta.reshape(1, OUT_DIM)
    fts = [_sc_gather(ints[i], table) for i in range(NSPLIT)]
    h = _TC_CALLS[0](fts[0], w_bf, b1r, g, be)
    for i in range(1, NSPLIT):
        h = _TC_CALLS[i](fts[i], w_bf, b1r, g, be, h)
    return h


# speedup vs baseline: 1.1054x; 1.1054x over previous
import functools

import jax
import jax.numpy as jnp
import numpy as np
from jax import lax
from jax.experimental import pallas as pl
from jax.experimental.pallas import tpu as pltpu
from jax.experimental.pallas import tpu_sc as plsc

NUM_FEATS = 26
SLOTS = 4
COLS = NUM_FEATS * SLOTS
COLS_PAD = 112
VOCAB = 1000
TAB_ROWS = NUM_FEATS * (VOCAB + 1)
EMB = 64
B = 4096
IN_DIM = NUM_FEATS * EMB
OUT_DIM = 4096
NUM_TOKENS = 8
D_MODEL = 512

NW = 32
NSPLIT = 4
BH = B // NSPLIT
ROWS_PER_W = BH // NW
CB = 8
NCHUNK = ROWS_PER_W // CB
JT = IN_DIM // 128

L = 16


def _sc_gather_body(ints_hbm, table_hbm, out_hbm, ints_v, idx_v, off_v, rows_v,
                    out_v, sems):
    cid = lax.axis_index("c")
    sid = lax.axis_index("s")
    wid = sid * 2 + cid
    lane = lax.iota(jnp.int32, L)
    colsv = jnp.full((L,), COLS, jnp.int32)

    for t in range(CB * COLS // L):
        j = lax.rem(t * L + lane, colsv)
        off_v[pl.ds(t * L, L)] = (j >> 2) * (VOCAB + 1)

    def fire(ch, buf):
        base = wid * ROWS_PER_W + ch * CB
        pltpu.sync_copy(ints_hbm.at[pl.ds(base * COLS, CB * COLS)],
                        ints_v.at[buf])
        for t in range(CB * COLS // L):
            sl = pl.ds(t * L, L)
            idx_v[buf, sl] = jnp.clip(ints_v[buf, sl], 0, VOCAB) + off_v[sl]
        pltpu.async_copy(table_hbm.at[idx_v.at[buf]],
                         rows_v.at[buf], sems.at[buf])

    def drain_reduce(ch, buf):
        pltpu.make_async_copy(table_hbm.at[idx_v.at[buf]],
                              rows_v.at[buf], sems.at[buf]).wait()

        def red_b(b, bcarry):
            def red_fp(fp, fcarry):
                for half in range(2):
                    rbase = b * COLS + 8 * fp + 4 * half
                    for c in range(EMB // L):
                        sl = pl.ds(c * L, L)
                        acc = (rows_v[buf, rbase, sl]
                               + rows_v[buf, rbase + 1, sl]
                               + rows_v[buf, rbase + 2, sl]
                               + rows_v[buf, rbase + 3, sl])
                        out_v[fp, b, pl.ds(64 * half + c * L, L)] = acc * 0.25
                return fcarry
            return lax.fori_loop(0, JT, red_fp, bcarry)

        lax.fori_loop(0, CB, red_b, 0)
        pltpu.sync_copy(out_v, out_hbm.at[wid * NCHUNK + ch])

    fire(0, 0)

    def chunk_body(ch, carry):
        buf = lax.rem(ch, 2)

        @pl.when(ch + 1 < NCHUNK)
        def _():
            fire(ch + 1, 1 - buf)

        drain_reduce(ch, buf)
        return carry

    lax.fori_loop(0, NCHUNK, chunk_body, 0)


@jax.jit
def _sc_gather(ints_flat, table_flat):
    mesh = plsc.VectorSubcoreMesh(core_axis_name="c", subcore_axis_name="s")
    return pl.kernel(
        _sc_gather_body,
        out_type=jax.ShapeDtypeStruct((BH // CB, JT, CB, 128), jnp.float32),
        mesh=mesh,
        scratch_types=[
            pltpu.VMEM((2, CB * COLS), jnp.int32),
            pltpu.VMEM((2, CB * COLS), jnp.int32),
            pltpu.VMEM((CB * COLS,), jnp.int32),
            pltpu.VMEM((2, CB * COLS, EMB), jnp.float32),
            pltpu.VMEM((JT, CB, 128), jnp.float32),
            pltpu.SemaphoreType.DMA((2,)),
        ],
        compiler_params=pltpu.CompilerParams(use_tc_tiling_on_sc=False),
    )(ints_flat, table_flat)


BT = 512
NT = 512
N_STEPS = OUT_DIM // NT


def _dense_body_alias(x_ref, w_ref, b_ref, g_ref, be_ref, h_ref, o_ref, acc_ref):
    _dense_body(x_ref, w_ref, b_ref, g_ref, be_ref, o_ref, acc_ref)


def _dense_body(x_ref, w_ref, b_ref, g_ref, be_ref, o_ref, acc_ref):
    n = pl.program_id(1)
    x = jnp.concatenate(
        [x_ref[:, j].reshape(BT, 128) for j in range(JT)], axis=1)
    h = jnp.dot(x.astype(jnp.bfloat16), w_ref[...],
                preferred_element_type=jnp.float32)
    h = h + b_ref[...]
    h = h * jax.nn.sigmoid(h)
    acc_ref[:, pl.ds(n * NT, NT)] = h

    @pl.when(n == N_STEPS - 1)
    def _():
        a = acc_ref[...]
        mu = jnp.mean(a, axis=1, keepdims=True)
        d = a - mu
        var = jnp.mean(d * d, axis=1, keepdims=True)
        out = d * lax.rsqrt(var + 1e-5) * g_ref[...] + be_ref[...]
        o_ref[...] = out.reshape(BT, NUM_TOKENS, D_MODEL)


def _make_tc_dense(tile_off, aliased):
    in_specs = [
        pl.BlockSpec((BT // CB, JT, CB, 128), lambda b, n: (b, 0, 0, 0)),
        pl.BlockSpec((IN_DIM, NT), lambda b, n: (0, n)),
        pl.BlockSpec((1, NT), lambda b, n: (0, n)),
        pl.BlockSpec((1, OUT_DIM), lambda b, n: (0, 0)),
        pl.BlockSpec((1, OUT_DIM), lambda b, n: (0, 0)),
    ]
    if aliased:
        in_specs.append(pl.BlockSpec(memory_space=pl.ANY))

    @jax.jit
    def call(*args):
        return pl.pallas_call(
            _dense_body_alias if aliased else _dense_body,
            grid=(BH // BT, N_STEPS),
            in_specs=in_specs,
            out_specs=pl.BlockSpec((BT, NUM_TOKENS, D_MODEL),
                                   lambda b, n: (b + tile_off, 0, 0)),
            out_shape=jax.ShapeDtypeStruct((B, NUM_TOKENS, D_MODEL),
                                           jnp.float32),
            scratch_shapes=[pltpu.VMEM((BT, OUT_DIM), jnp.float32)],
            input_output_aliases={5: 0} if aliased else {},
            compiler_params=pltpu.CompilerParams(
                dimension_semantics=("parallel", "arbitrary"),
            ),
        )(*args)

    return call


_TC_CALLS = [_make_tc_dense(i * (BH // BT), aliased=(i > 0))
             for i in range(NSPLIT)]


def kernel(int_feats, emb_tables, W1, b1, ln_gamma, ln_beta):
    ints = int_feats.astype(jnp.int32).reshape(NSPLIT, BH * COLS)
    table = emb_tables.reshape(TAB_ROWS, EMB)
    w_bf = W1.astype(jnp.bfloat16)
    b1r = b1.reshape(1, OUT_DIM)
    g = ln_gamma.reshape(1, OUT_DIM)
    be = ln_beta.reshape(1, OUT_DIM)
    fts = [_sc_gather(ints[i], table) for i in range(NSPLIT)]
    h = _TC_CALLS[0](fts[0], w_bf, b1r, g, be)
    for i in range(1, NSPLIT):
        h = _TC_CALLS[i](fts[i], w_bf, b1r, g, be, h)
    return h
